# baseline (speedup 1.0000x reference)
import functools

import jax
import jax.numpy as jnp
from jax import lax
from jax.experimental import pallas as pl
from jax.experimental.pallas import tpu as pltpu
from jax.experimental.pallas import tpu_sc as plsc

E = 8
NEG_INF = float("-inf")


def _logits_kernel(x_ref, wg_ref, out_ref):
    lt = jax.lax.dot_general(
        wg_ref[...], x_ref[...], (((1,), (1,)), ((), ())),
        preferred_element_type=jnp.float32)
    out_ref[...] = lt


def _make_combine_sc(t_tokens):
    info = plsc.get_sparse_core_info()
    nc, ns, lanes = info.num_cores, info.num_subcores, info.num_lanes
    nw = nc * ns
    per_w = t_tokens // nw
    n_chunks = per_w // lanes
    mesh = plsc.VectorSubcoreMesh(core_axis_name="c", subcore_axis_name="s")

    @functools.partial(
        pl.kernel, mesh=mesh,
        out_type=jax.ShapeDtypeStruct((E, t_tokens), jnp.float32),
        scratch_types=[
            pltpu.VMEM((E, per_w), jnp.float32),
            pltpu.VMEM((E, per_w), jnp.float32),
        ],
    )
    def combine_sc(lt_hbm, out_hbm, lt_v, cmb_v):
        wid = lax.axis_index("s") * nc + lax.axis_index("c")
        base = wid * per_w
        pltpu.sync_copy(lt_hbm.at[:, pl.ds(base, per_w)], lt_v)

        def chunk(k, carry):
            del carry
            off = k * lanes
            rows = [lt_v[e, pl.ds(off, lanes)] for e in range(E)]
            l1 = rows[0]
            i1 = jnp.zeros((lanes,), jnp.int32)
            for e in range(1, E):
                gt = rows[e] > l1
                l1 = jnp.where(gt, rows[e], l1)
                i1 = jnp.where(gt, e, i1)
            l2 = jnp.full((lanes,), NEG_INF, jnp.float32)
            i2 = jnp.zeros((lanes,), jnp.int32)
            for e in range(E):
                valid = i1 != e
                gt = valid & (rows[e] > l2)
                l2 = jnp.where(gt, rows[e], l2)
                i2 = jnp.where(gt, e, i2)
            e21 = jnp.exp(l2 - l1)
            w2 = e21 / (1.0 + e21)
            w1 = 1.0 - w2
            for e in range(E):
                ce = (jnp.where(i1 == e, w1, 0.0)
                      + jnp.where(i2 == e, w2, 0.0))
                cmb_v[e, pl.ds(off, lanes)] = ce
            return 0

        lax.fori_loop(0, n_chunks, chunk, 0)
        pltpu.sync_copy(cmb_v, out_hbm.at[:, pl.ds(base, per_w)])

    return combine_sc


def _moe_tile_kernel(x_ref, cmb_ref, we_ref, be_ref, out_ref, wb_ref):
    @pl.when(pl.program_id(0) == 0)
    def _cast_weights():
        wb_ref[...] = we_ref[...].astype(jnp.bfloat16)

    x = x_ref[...]
    tile = x.shape[0]
    combine = cmb_ref[...]
    acc = jax.lax.dot_general(
        combine, be_ref[...], (((0,), (0,)), ((), ())),
        preferred_element_type=jnp.float32)
    for e in range(E):
        y = jax.lax.dot_general(
            x, wb_ref[e], (((1,), (1,)), ((), ())),
            preferred_element_type=jnp.float32)
        acc += combine[e, :][:, None] * y
    out_ref[...] = acc


@functools.partial(jax.jit, static_argnames=())
def kernel(inputs, W_gate, W_experts, b_experts):
    batch_shape = inputs.shape[:-1]
    d_in = inputs.shape[-1]
    x = inputs.reshape(-1, d_in)
    t = x.shape[0]
    d_out = W_experts.shape[1]
    tile = 1024
    grid = (t // tile,)

    logits_t = pl.pallas_call(
        _logits_kernel,
        grid=grid,
        in_specs=[
            pl.BlockSpec((tile, d_in), lambda i: (i, 0)),
            pl.BlockSpec((E, d_in), lambda i: (0, 0)),
        ],
        out_specs=pl.BlockSpec((E, tile), lambda i: (0, i)),
        out_shape=jax.ShapeDtypeStruct((E, t), jnp.float32),
    )(x, W_gate)

    combine_t = _make_combine_sc(t)(logits_t)

    out = pl.pallas_call(
        _moe_tile_kernel,
        grid=grid,
        in_specs=[
            pl.BlockSpec((tile, d_in), lambda i: (i, 0)),
            pl.BlockSpec((E, tile), lambda i: (0, i)),
            pl.BlockSpec((E, d_out, d_in), lambda i: (0, 0, 0)),
            pl.BlockSpec((E, d_out), lambda i: (0, 0)),
        ],
        out_specs=pl.BlockSpec((tile, d_out), lambda i: (i, 0)),
        out_shape=jax.ShapeDtypeStruct((t, d_out), jnp.float32),
        scratch_shapes=[pltpu.VMEM((E, d_out, d_in), jnp.bfloat16)],
    )(x, combine_t, W_experts, b_experts)
    return out.reshape(*batch_shape, d_out)


# speedup vs baseline: 1.1428x; 1.1428x over previous
import functools

import jax
import jax.numpy as jnp
from jax.experimental import pallas as pl
from jax.experimental.pallas import tpu as pltpu

E = 8
TOP_K = 2
NEG_INF = float("-inf")


def _moe_tile_kernel(x_ref, wg_ref, we_hbm, be_ref, out_ref, w_vmem, sem1, sem2):
    i = pl.program_id(0)
    h = E // 2

    @pl.when(i == 0)
    def _start_w_copies():
        pltpu.make_async_copy(we_hbm.at[pl.ds(0, h)], w_vmem.at[pl.ds(0, h)], sem1).start()
        pltpu.make_async_copy(we_hbm.at[pl.ds(h, h)], w_vmem.at[pl.ds(h, h)], sem2).start()

    x = x_ref[...]
    tile = x.shape[0]

    logits = jax.lax.dot_general(
        x, wg_ref[...], (((1,), (1,)), ((), ())),
        preferred_element_type=jnp.float32)
    eids = jax.lax.broadcasted_iota(jnp.int32, (tile, E), 1)
    l1 = jnp.max(logits, axis=1, keepdims=True)
    i1 = jnp.min(jnp.where(logits == l1, eids, E), axis=1, keepdims=True)
    masked = jnp.where(eids == i1, NEG_INF, logits)
    l2 = jnp.max(masked, axis=1, keepdims=True)
    i2 = jnp.min(jnp.where(masked == l2, eids, E), axis=1, keepdims=True)
    e21 = jnp.exp(l2 - l1)
    w2 = e21 / (1.0 + e21)
    w1 = 1.0 - w2
    combine = jnp.where(eids == i1, w1, 0.0) + jnp.where(eids == i2, w2, 0.0)

    acc = jax.lax.dot_general(
        combine, be_ref[...], (((1,), (0,)), ((), ())),
        preferred_element_type=jnp.float32)
    @pl.when(i == 0)
    def _wait_first_half():
        pltpu.make_async_copy(we_hbm.at[pl.ds(0, h)], w_vmem.at[pl.ds(0, h)], sem1).wait()

    for e in range(E):
        if e == h:
            @pl.when(i == 0)
            def _wait_second_half():
                pltpu.make_async_copy(we_hbm.at[pl.ds(h, h)], w_vmem.at[pl.ds(h, h)], sem2).wait()
        y = jax.lax.dot_general(
            x, w_vmem[e], (((1,), (1,)), ((), ())),
            preferred_element_type=jnp.float32)
        acc += combine[:, e][:, None] * y
    out_ref[...] = acc


@functools.partial(jax.jit, static_argnames=())
def kernel(inputs, W_gate, W_experts, b_experts):
    batch_shape = inputs.shape[:-1]
    d_in = inputs.shape[-1]
    x = inputs.reshape(-1, d_in)
    t = x.shape[0]
    d_out = W_experts.shape[1]
    tile = 1024
    grid = (t // tile,)

    out = pl.pallas_call(
        _moe_tile_kernel,
        grid=grid,
        in_specs=[
            pl.BlockSpec((tile, d_in), lambda i: (i, 0)),
            pl.BlockSpec((E, d_in), lambda i: (0, 0)),
            pl.BlockSpec(memory_space=pl.ANY),
            pl.BlockSpec((E, d_out), lambda i: (0, 0)),
        ],
        out_specs=pl.BlockSpec((tile, d_out), lambda i: (i, 0)),
        out_shape=jax.ShapeDtypeStruct((t, d_out), jnp.float32),
        scratch_shapes=[
            pltpu.VMEM((E, d_out, d_in), jnp.float32),
            pltpu.SemaphoreType.DMA,
            pltpu.SemaphoreType.DMA,
        ],
    )(x, W_gate, W_experts, b_experts)
    return out.reshape(*batch_shape, d_out)


# speedup vs baseline: 1.2727x; 1.1136x over previous
import functools

import jax
import jax.numpy as jnp
from jax.experimental import pallas as pl
from jax.experimental.pallas import tpu as pltpu

E = 8
TOP_K = 2
NEG_INF = float("-inf")


def _moe_tile_kernel(x_ref, wg_ref, we_ref, be_ref, out_ref):
    x = x_ref[...]
    tile = x.shape[0]

    logits = jax.lax.dot_general(
        x, wg_ref[...], (((1,), (1,)), ((), ())),
        preferred_element_type=jnp.float32)
    eids = jax.lax.broadcasted_iota(jnp.int32, (tile, E), 1)
    l1 = jnp.max(logits, axis=1, keepdims=True)
    i1 = jnp.min(jnp.where(logits == l1, eids, E), axis=1, keepdims=True)
    masked = jnp.where(eids == i1, NEG_INF, logits)
    l2 = jnp.max(masked, axis=1, keepdims=True)
    i2 = jnp.min(jnp.where(masked == l2, eids, E), axis=1, keepdims=True)
    e21 = jnp.exp(l2 - l1)
    w2 = e21 / (1.0 + e21)
    w1 = 1.0 - w2
    combine = jnp.where(eids == i1, w1, 0.0) + jnp.where(eids == i2, w2, 0.0)

    acc = jax.lax.dot_general(
        combine, be_ref[...], (((1,), (0,)), ((), ())),
        preferred_element_type=jnp.float32)
    for e in range(E):
        y = jax.lax.dot_general(
            x, we_ref[e], (((1,), (1,)), ((), ())),
            preferred_element_type=jnp.float32)
        acc += combine[:, e][:, None] * y
    out_ref[...] = acc


@functools.partial(jax.jit, static_argnames=())
def kernel(inputs, W_gate, W_experts, b_experts):
    batch_shape = inputs.shape[:-1]
    d_in = inputs.shape[-1]
    x = inputs.reshape(-1, d_in)
    t = x.shape[0]
    d_out = W_experts.shape[1]
    tile = 1024
    grid = (t // tile,)

    out = pl.pallas_call(
        _moe_tile_kernel,
        grid=grid,
        in_specs=[
            pl.BlockSpec((tile, d_in), lambda i: (i, 0)),
            pl.BlockSpec((E, d_in), lambda i: (0, 0)),
            pl.BlockSpec((E, d_out, d_in), lambda i: (0, 0, 0)),
            pl.BlockSpec((E, d_out), lambda i: (0, 0)),
        ],
        out_specs=pl.BlockSpec((tile, d_out), lambda i: (i, 0)),
        out_shape=jax.ShapeDtypeStruct((t, d_out), jnp.float32),
    )(x, W_gate, W_experts, b_experts)
    return out.reshape(*batch_shape, d_out)
